# Initial kernel scaffold; baseline (speedup 1.0000x reference)
import jax
import jax.numpy as jnp
from jax.experimental import pallas as pl


def kernel(query_xyz, query_quats, scales, opacities, sph, sk_ind, sk_w, node_ref_xyz, node_ref_quat, node_tgt_xyz, node_tgt_quat):
    raise NotImplementedError("write your pallas kernel here")


# speedup vs baseline: 12.1468x; 12.1468x over previous
import functools

import jax
import jax.numpy as jnp
from jax import lax
from jax.experimental import pallas as pl
from jax.experimental.pallas import tpu as pltpu
from jax.experimental.pallas import tpu_sc as plsc

_NC = 2
_NS = 16
_NW = _NC * _NS
_L = 16
_CHUNK = 512


def _rsqrt(x):
    i = plsc.bitcast(x, jnp.int32)
    y = plsc.bitcast(jnp.int32(0x5F3759DF) - (i >> 1), jnp.float32)
    for _ in range(3):
        y = y * (1.5 - 0.5 * x * y * y)
    return y


def _inv_norm4(w, x, y, z):
    n2 = w * w + x * x + y * y + z * z
    nrm = n2 * _rsqrt(jnp.maximum(n2, 1e-30))
    return 1.0 / (nrm + 1e-8)


def _rotmat(w, x, y, z):
    inv = _inv_norm4(w, x, y, z)
    w, x, y, z = w * inv, x * inv, y * inv, z * inv
    x2, y2, z2 = x + x, y + y, z + z
    xx, yy, zz = x2 * x, y2 * y, z2 * z
    xy, xz, yz = x2 * y, x2 * z, y2 * z
    wx, wy, wz = x2 * w, y2 * w, z2 * w
    return ((1.0 - (yy + zz), xy - wz, xz + wy),
            (xy + wz, 1.0 - (xx + zz), yz - wx),
            (xz - wy, yz + wx, 1.0 - (xx + yy)))


@functools.lru_cache(maxsize=None)
def _build(N, M):
    assert N % (_NW * _CHUNK) == 0 and M % (_NS * _L) == 0
    G = N // _NW
    NCH = G // _CHUNK
    GROUPS = _CHUNK // _L
    MSL = M // _NS

    mesh = plsc.VectorSubcoreMesh(core_axis_name="c", subcore_axis_name="s")
    f32 = jnp.float32

    @functools.partial(
        pl.kernel,
        out_type=(
            jax.ShapeDtypeStruct((N * 3,), f32),
            jax.ShapeDtypeStruct((N * 9,), f32),
            jax.ShapeDtypeStruct((N * 3,), f32),
            jax.ShapeDtypeStruct((N,), f32),
        ),
        mesh=mesh,
        compiler_params=pltpu.CompilerParams(
            needs_layout_passes=False,
            use_tc_tiling_on_sc=False,
        ),
        scratch_types=(
            pltpu.VMEM_SHARED((7 * M,), f32),
            pltpu.VMEM((7 * M,), f32),
            pltpu.VMEM((MSL * 3,), f32),
            pltpu.VMEM((MSL * 4,), f32),
            pltpu.VMEM((MSL * 3,), f32),
            pltpu.VMEM((MSL * 4,), f32),
            pltpu.VMEM((7 * MSL,), f32),
            pltpu.VMEM((_CHUNK * 3,), f32),
            pltpu.VMEM((_CHUNK * 4,), f32),
            pltpu.VMEM((_CHUNK * 3,), f32),
            pltpu.VMEM((_CHUNK,), f32),
            pltpu.VMEM((_CHUNK * 8,), jnp.int32),
            pltpu.VMEM((_CHUNK * 8,), f32),
            pltpu.VMEM((_CHUNK * 3,), f32),
            pltpu.VMEM((_CHUNK * 9,), f32),
            pltpu.VMEM((_CHUNK * 3,), f32),
            pltpu.VMEM((_CHUNK,), f32),
        ),
    )
    def skin(qx_h, qq_h, sc_h, op_h, ind_h, w_h,
             nrx_h, nrq_h, ntx_h, ntq_h,
             mu_h, fr_h, s_h, o_h,
             shared, table, nrx_s, nrq_s, ntx_s, ntq_s, tsl,
             cxyz, cqt, csc, cop, cind, cw, cmu, cfr, cs, co):
        ci = lax.axis_index("c")
        si = lax.axis_index("s")
        wid = si * _NC + ci

        iota = jnp.arange(_L, dtype=jnp.int32)
        i3 = iota * 3
        i4 = iota * 4
        i8 = iota * 8
        i9 = iota * 9

        pltpu.sync_copy(nrx_h.at[pl.ds(si * (MSL * 3), MSL * 3)], nrx_s)
        pltpu.sync_copy(nrq_h.at[pl.ds(si * (MSL * 4), MSL * 4)], nrq_s)
        pltpu.sync_copy(ntx_h.at[pl.ds(si * (MSL * 3), MSL * 3)], ntx_s)
        pltpu.sync_copy(ntq_h.at[pl.ds(si * (MSL * 4), MSL * 4)], ntq_s)

        def node_group(g, carry):
            rq = [plsc.load_gather(nrq_s, [i4 + (g * (4 * _L) + cc)])
                  for cc in range(4)]
            tq = [plsc.load_gather(ntq_s, [i4 + (g * (4 * _L) + cc)])
                  for cc in range(4)]
            rv = [plsc.load_gather(nrx_s, [i3 + (g * (3 * _L) + cc)])
                  for cc in range(3)]
            tv = [plsc.load_gather(ntx_s, [i3 + (g * (3 * _L) + cc)])
                  for cc in range(3)]
            rinv = _inv_norm4(*rq)
            tinv = _inv_norm4(*tq)
            aw, ax, ay, az = (q * tinv for q in tq)
            bw, bx, by, bz = rq[0] * rinv, -rq[1] * rinv, -rq[2] * rinv, -rq[3] * rinv
            dw = aw * bw - ax * bx - ay * by - az * bz
            dx = aw * bx + ax * bw + ay * bz - az * by
            dy = aw * by - ax * bz + ay * bw + az * bx
            dz = aw * bz + ax * by - ay * bx + az * bw
            R = _rotmat(dw, dx, dy, dz)
            t = [tv[r] - (R[r][0] * rv[0] + R[r][1] * rv[1] + R[r][2] * rv[2])
                 for r in range(3)]
            base = g * _L
            tsl[pl.ds(0 * MSL + base, _L)] = dw
            tsl[pl.ds(1 * MSL + base, _L)] = dx
            tsl[pl.ds(2 * MSL + base, _L)] = dy
            tsl[pl.ds(3 * MSL + base, _L)] = dz
            tsl[pl.ds(4 * MSL + base, _L)] = t[0]
            tsl[pl.ds(5 * MSL + base, _L)] = t[1]
            tsl[pl.ds(6 * MSL + base, _L)] = t[2]
            return carry

        lax.fori_loop(0, MSL // _L, node_group, 0)
        for comp in range(7):
            pltpu.sync_copy(tsl.at[pl.ds(comp * MSL, MSL)],
                            shared.at[pl.ds(comp * M + si * MSL, MSL)])
        plsc.subcore_barrier()
        pltpu.sync_copy(shared, table)

        g0 = wid * G

        def chunk_fn(cb, carry):
            b = g0 + cb * _CHUNK
            pltpu.sync_copy(qx_h.at[pl.ds(b * 3, _CHUNK * 3)], cxyz)
            pltpu.sync_copy(qq_h.at[pl.ds(b * 4, _CHUNK * 4)], cqt)
            pltpu.sync_copy(sc_h.at[pl.ds(b * 3, _CHUNK * 3)], csc)
            pltpu.sync_copy(op_h.at[pl.ds(b, _CHUNK)], cop)
            pltpu.sync_copy(ind_h.at[pl.ds(b * 8, _CHUNK * 8)], cind)
            pltpu.sync_copy(w_h.at[pl.ds(b * 8, _CHUNK * 8)], cw)

            def group_fn(j, carry2):
                ks = [plsc.load_gather(cind, [i8 + (j * (8 * _L) + k)])
                      for k in range(8)]
                ws = [plsc.load_gather(cw, [i8 + (j * (8 * _L) + k)])
                      for k in range(8)]
                wsum = ws[0]
                for k in range(1, 8):
                    wsum = wsum + ws[k]
                winv = 1.0 / (wsum + 1e-8)

                q0 = [plsc.load_gather(table, [ks[0] + cc * M])
                      for cc in range(4)]
                aq = [ws[0] * q0[cc] for cc in range(4)]
                at = [ws[0] * plsc.load_gather(table, [ks[0] + (4 + cc) * M])
                      for cc in range(3)]
                for k in range(1, 8):
                    qk = [plsc.load_gather(table, [ks[k] + cc * M])
                          for cc in range(4)]
                    d = (q0[0] * qk[0] + q0[1] * qk[1]
                         + q0[2] * qk[2] + q0[3] * qk[3])
                    wk = jnp.where(d < 0, -ws[k], ws[k])
                    for cc in range(4):
                        aq[cc] = aq[cc] + wk * qk[cc]
                    for cc in range(3):
                        at[cc] = at[cc] + ws[k] * plsc.load_gather(
                            table, [ks[k] + (4 + cc) * M])

                qb = [a * winv for a in aq]
                tb = [a * winv for a in at]
                Rb = _rotmat(qb[0], qb[1], qb[2], qb[3])

                v = [plsc.load_gather(cxyz, [i3 + (j * (3 * _L) + cc)])
                     for cc in range(3)]
                for r in range(3):
                    mur = (Rb[r][0] * v[0] + Rb[r][1] * v[1]
                           + Rb[r][2] * v[2] + tb[r])
                    plsc.store_scatter(cmu, [i3 + (j * (3 * _L) + r)], mur)

                qr = [plsc.load_gather(cqt, [i4 + (j * (4 * _L) + cc)])
                      for cc in range(4)]
                Rr = _rotmat(qr[0], qr[1], qr[2], qr[3])
                for r in range(3):
                    for col in range(3):
                        fr = (Rb[r][0] * Rr[0][col] + Rb[r][1] * Rr[1][col]
                              + Rb[r][2] * Rr[2][col])
                        plsc.store_scatter(
                            cfr, [i9 + (j * (9 * _L) + (3 * r + col))], fr)

                for cc in range(3):
                    sv = plsc.load_gather(csc, [i3 + (j * (3 * _L) + cc)])
                    plsc.store_scatter(cs, [i3 + (j * (3 * _L) + cc)],
                                       jnp.exp(sv))

                ov = plsc.load_gather(cop, [iota + j * _L])
                plsc.store_scatter(co, [iota + j * _L],
                                   1.0 / (1.0 + jnp.exp(-ov)))
                return carry2

            lax.fori_loop(0, GROUPS, group_fn, 0)

            pltpu.sync_copy(cmu, mu_h.at[pl.ds(b * 3, _CHUNK * 3)])
            pltpu.sync_copy(cfr, fr_h.at[pl.ds(b * 9, _CHUNK * 9)])
            pltpu.sync_copy(cs, s_h.at[pl.ds(b * 3, _CHUNK * 3)])
            pltpu.sync_copy(co, o_h.at[pl.ds(b, _CHUNK)])
            return carry

        lax.fori_loop(0, NCH, chunk_fn, 0)

    return skin


def kernel(query_xyz, query_quats, scales, opacities, sph, sk_ind, sk_w,
           node_ref_xyz, node_ref_quat, node_tgt_xyz, node_tgt_quat):
    N = query_xyz.shape[0]
    M = node_ref_xyz.shape[0]
    assert sk_ind.shape[1] == 8
    mu_f, fr_f, s_f, o_f = _build(N, M)(
        query_xyz.reshape(-1),
        query_quats.reshape(-1),
        scales.reshape(-1),
        opacities,
        sk_ind.astype(jnp.int32).reshape(-1),
        sk_w.reshape(-1),
        node_ref_xyz.reshape(-1),
        node_ref_quat.reshape(-1),
        node_tgt_xyz.reshape(-1),
        node_tgt_quat.reshape(-1),
    )
    return (mu_f.reshape(N, 3), fr_f.reshape(N, 3, 3),
            s_f.reshape(N, 3), o_f, sph)


# speedup vs baseline: 84.0482x; 6.9194x over previous
import functools

import jax
import jax.numpy as jnp
from jax import lax
from jax.experimental import pallas as pl
from jax.experimental.pallas import tpu as pltpu
from jax.experimental.pallas import tpu_sc as plsc

_NC = 2
_NS = 16
_NW = _NC * _NS
_L = 16
_CHUNK = 512


def _rsqrt(x):
    i = plsc.bitcast(x, jnp.int32)
    y = plsc.bitcast(jnp.int32(0x5F3759DF) - (i >> 1), jnp.float32)
    for _ in range(3):
        y = y * (1.5 - 0.5 * x * y * y)
    return y


def _inv_norm4(w, x, y, z):
    n2 = w * w + x * x + y * y + z * z
    nrm = n2 * _rsqrt(jnp.maximum(n2, 1e-30))
    return 1.0 / (nrm + 1e-8)


def _rotmat(w, x, y, z):
    inv = _inv_norm4(w, x, y, z)
    w, x, y, z = w * inv, x * inv, y * inv, z * inv
    x2, y2, z2 = x + x, y + y, z + z
    xx, yy, zz = x2 * x, y2 * y, z2 * z
    xy, xz, yz = x2 * y, x2 * z, y2 * z
    wx, wy, wz = x2 * w, y2 * w, z2 * w
    return ((1.0 - (yy + zz), xy - wz, xz + wy),
            (xy + wz, 1.0 - (xx + zz), yz - wx),
            (xz - wy, yz + wx, 1.0 - (xx + yy)))


@functools.lru_cache(maxsize=None)
def _build(N, M):
    assert N % (_NW * _CHUNK) == 0 and M % (_NS * _L) == 0
    G = N // _NW
    NCH = G // _CHUNK
    GROUPS = _CHUNK // _L
    MSL = M // _NS

    mesh = plsc.VectorSubcoreMesh(core_axis_name="c", subcore_axis_name="s")
    f32 = jnp.float32

    @functools.partial(
        pl.kernel,
        out_type=(
            jax.ShapeDtypeStruct((3 * N,), f32),
            jax.ShapeDtypeStruct((9 * N,), f32),
            jax.ShapeDtypeStruct((3 * N,), f32),
            jax.ShapeDtypeStruct((N,), f32),
        ),
        mesh=mesh,
        compiler_params=pltpu.CompilerParams(
            needs_layout_passes=False,
            use_tc_tiling_on_sc=False,
        ),
        scratch_types=(
            pltpu.VMEM_SHARED((7 * M,), f32),
            pltpu.VMEM((7 * M,), f32),
            pltpu.VMEM((14 * MSL,), f32),
            pltpu.VMEM((7 * MSL,), f32),
            pltpu.VMEM((19 * _CHUNK,), f32),
            pltpu.VMEM((8 * _CHUNK,), jnp.int32),
            pltpu.VMEM((16 * _CHUNK,), f32),
            pltpu.SemaphoreType.DMA,
            pltpu.SemaphoreType.DMA,
        ),
    )
    def skin(qx_h, qq_h, sc_h, op_h, ind_h, w_h,
             nrx_h, nrq_h, ntx_h, ntq_h,
             mu_h, fr_h, s_h, o_h,
             shared, table, nin, tsl, cfin, cind, cfout, isem, osem):
        ci = lax.axis_index("c")
        si = lax.axis_index("s")
        wid = si * _NC + ci

        iota = jnp.arange(_L, dtype=jnp.int32)

        nb = si * MSL
        handles = []
        for r, ncomp, src in ((0, 3, nrx_h), (3, 4, nrq_h),
                              (7, 3, ntx_h), (10, 4, ntq_h)):
            for cc in range(ncomp):
                handles.append(pltpu.async_copy(
                    src.at[pl.ds(cc * M + nb, MSL)],
                    nin.at[pl.ds((r + cc) * MSL, MSL)], isem))
        for h in handles:
            h.wait()

        def node_group(g, carry):
            def ld(row):
                return plsc.load_gather(nin, [iota + (row * MSL + g * _L)])
            rv = [ld(0), ld(1), ld(2)]
            rq = [ld(3), ld(4), ld(5), ld(6)]
            tv = [ld(7), ld(8), ld(9)]
            tq = [ld(10), ld(11), ld(12), ld(13)]
            rinv = _inv_norm4(*rq)
            tinv = _inv_norm4(*tq)
            aw, ax, ay, az = (q * tinv for q in tq)
            bw = rq[0] * rinv
            bx = -rq[1] * rinv
            by = -rq[2] * rinv
            bz = -rq[3] * rinv
            dw = aw * bw - ax * bx - ay * by - az * bz
            dx = aw * bx + ax * bw + ay * bz - az * by
            dy = aw * by - ax * bz + ay * bw + az * bx
            dz = aw * bz + ax * by - ay * bx + az * bw
            R = _rotmat(dw, dx, dy, dz)
            t = [tv[r] - (R[r][0] * rv[0] + R[r][1] * rv[1] + R[r][2] * rv[2])
                 for r in range(3)]
            base = g * _L
            for row, val in enumerate((dw, dx, dy, dz, t[0], t[1], t[2])):
                plsc.store_scatter(tsl, [iota + (row * MSL + base)], val)
            return carry

        lax.fori_loop(0, MSL // _L, node_group, 0)
        for comp in range(7):
            pltpu.sync_copy(tsl.at[pl.ds(comp * MSL, MSL)],
                            shared.at[pl.ds(comp * M + si * MSL, MSL)])
        plsc.subcore_barrier()
        pltpu.sync_copy(shared, table)

        g0 = wid * G

        def chunk_fn(cb, carry):
            b = g0 + cb * _CHUNK
            hs = []
            for cc in range(3):
                hs.append(pltpu.async_copy(
                    qx_h.at[pl.ds(cc * N + b, _CHUNK)],
                    cfin.at[pl.ds(cc * _CHUNK, _CHUNK)], isem))
            for cc in range(4):
                hs.append(pltpu.async_copy(
                    qq_h.at[pl.ds(cc * N + b, _CHUNK)],
                    cfin.at[pl.ds((3 + cc) * _CHUNK, _CHUNK)], isem))
            for cc in range(3):
                hs.append(pltpu.async_copy(
                    sc_h.at[pl.ds(cc * N + b, _CHUNK)],
                    cfin.at[pl.ds((7 + cc) * _CHUNK, _CHUNK)], isem))
            for cc in range(8):
                hs.append(pltpu.async_copy(
                    w_h.at[pl.ds(cc * N + b, _CHUNK)],
                    cfin.at[pl.ds((10 + cc) * _CHUNK, _CHUNK)], isem))
            hs.append(pltpu.async_copy(
                op_h.at[pl.ds(b, _CHUNK)],
                cfin.at[pl.ds(18 * _CHUNK, _CHUNK)], isem))
            for cc in range(8):
                hs.append(pltpu.async_copy(
                    ind_h.at[pl.ds(cc * N + b, _CHUNK)],
                    cind.at[pl.ds(cc * _CHUNK, _CHUNK)], isem))
            for h in hs:
                h.wait()

            def group_fn(j, carry2):
                base = j * _L

                def ldf(row):
                    return plsc.load_gather(
                        cfin, [iota + (row * _CHUNK + base)])

                ks = [plsc.load_gather(cind, [iota + (k * _CHUNK + base)])
                      for k in range(8)]
                ws = [ldf(10 + k) for k in range(8)]
                wsum = ws[0]
                for k in range(1, 8):
                    wsum = wsum + ws[k]
                winv = 1.0 / (wsum + 1e-8)

                q0 = [plsc.load_gather(table, [ks[0] + cc * M])
                      for cc in range(4)]
                aq = [ws[0] * q0[cc] for cc in range(4)]
                at = [ws[0] * plsc.load_gather(table, [ks[0] + (4 + cc) * M])
                      for cc in range(3)]
                for k in range(1, 8):
                    qk = [plsc.load_gather(table, [ks[k] + cc * M])
                          for cc in range(4)]
                    d = (q0[0] * qk[0] + q0[1] * qk[1]
                         + q0[2] * qk[2] + q0[3] * qk[3])
                    wk = jnp.where(d < 0, -ws[k], ws[k])
                    for cc in range(4):
                        aq[cc] = aq[cc] + wk * qk[cc]
                    for cc in range(3):
                        at[cc] = at[cc] + ws[k] * plsc.load_gather(
                            table, [ks[k] + (4 + cc) * M])

                qb = [a * winv for a in aq]
                tb = [a * winv for a in at]
                Rb = _rotmat(qb[0], qb[1], qb[2], qb[3])

                def stf(row, val):
                    plsc.store_scatter(
                        cfout, [iota + (row * _CHUNK + base)], val)

                v = [ldf(0), ldf(1), ldf(2)]
                for r in range(3):
                    stf(r, Rb[r][0] * v[0] + Rb[r][1] * v[1]
                        + Rb[r][2] * v[2] + tb[r])

                qr = [ldf(3), ldf(4), ldf(5), ldf(6)]
                Rr = _rotmat(qr[0], qr[1], qr[2], qr[3])
                for r in range(3):
                    for col in range(3):
                        stf(3 + 3 * r + col,
                            Rb[r][0] * Rr[0][col] + Rb[r][1] * Rr[1][col]
                            + Rb[r][2] * Rr[2][col])

                for cc in range(3):
                    stf(12 + cc, jnp.exp(ldf(7 + cc)))
                stf(15, 1.0 / (1.0 + jnp.exp(-ldf(18))))
                return carry2

            lax.fori_loop(0, GROUPS, group_fn, 0)

            os_ = []
            for r in range(3):
                os_.append(pltpu.async_copy(
                    cfout.at[pl.ds(r * _CHUNK, _CHUNK)],
                    mu_h.at[pl.ds(r * N + b, _CHUNK)], osem))
            for r in range(9):
                os_.append(pltpu.async_copy(
                    cfout.at[pl.ds((3 + r) * _CHUNK, _CHUNK)],
                    fr_h.at[pl.ds(r * N + b, _CHUNK)], osem))
            for r in range(3):
                os_.append(pltpu.async_copy(
                    cfout.at[pl.ds((12 + r) * _CHUNK, _CHUNK)],
                    s_h.at[pl.ds(r * N + b, _CHUNK)], osem))
            os_.append(pltpu.async_copy(
                cfout.at[pl.ds(15 * _CHUNK, _CHUNK)],
                o_h.at[pl.ds(b, _CHUNK)], osem))
            for h in os_:
                h.wait()
            return carry

        lax.fori_loop(0, NCH, chunk_fn, 0)

    return skin


def kernel(query_xyz, query_quats, scales, opacities, sph, sk_ind, sk_w,
           node_ref_xyz, node_ref_quat, node_tgt_xyz, node_tgt_quat):
    N = query_xyz.shape[0]
    M = node_ref_xyz.shape[0]
    assert sk_ind.shape[1] == 8
    mu_t, fr_t, s_t, o = _build(N, M)(
        query_xyz.T.reshape(-1),
        query_quats.T.reshape(-1),
        scales.T.reshape(-1),
        opacities,
        sk_ind.astype(jnp.int32).T.reshape(-1),
        sk_w.T.reshape(-1),
        node_ref_xyz.T.reshape(-1),
        node_ref_quat.T.reshape(-1),
        node_tgt_xyz.T.reshape(-1),
        node_tgt_quat.T.reshape(-1),
    )
    mu = mu_t.reshape(3, N).T
    fr = fr_t.reshape(3, 3, N).transpose(2, 0, 1)
    s = s_t.reshape(3, N).T
    return (mu, fr, s, o, sph)
